# Initial kernel scaffold; baseline (speedup 1.0000x reference)
import jax
import jax.numpy as jnp
from jax.experimental import pallas as pl


def kernel(x, edge_index, batch, eps1, w1a, b1a, g1, be1, rm1, rv1, w1b, b1b, eps2, w2a, b2a, g2, be2, rm2, rv2, w2b, b2b, eps3, w3a, b3a, g3, be3, rm3, rv3, w3b, b3b, wc, bc):
    raise NotImplementedError("write your pallas kernel here")


# speedup vs baseline: 8.2662x; 8.2662x over previous
import functools

import jax
import jax.numpy as jnp
from jax import lax
from jax.experimental import pallas as pl
from jax.experimental.pallas import tpu as pltpu
from jax.experimental.pallas import tpu_sc as plsc

N = 10000
E = 320000
D = 128
G = 64

NC = 2
NS = 16
NW = NC * NS
EPW = E // NW
CH = 80
NCH = EPW // CH
IB = 25
NIB = NCH // IB
NP = 10240
RPT = NP // NS

RB = 1000
NRB = N // RB


def _sc_agg_body(x_hbm, src_hbm, dst_hbm, out_hbm,
                 shared, src_v, dst_v, rows_a, rows_b, sem_a, sem_b):
    c = lax.axis_index("c")
    s = lax.axis_index("s")
    w = c * NS + s

    def zrow(i, carry):
        for lb in range(D // 16):
            rows_a[i, pl.ds(lb * 16, 16)] = jnp.zeros((16,), jnp.float32)
        return carry
    lax.fori_loop(0, CH, zrow, 0)
    for k in range(RPT // CH):
        pltpu.sync_copy(rows_a, shared.at[pl.ds(s * RPT + k * CH, CH)])
    plsc.subcore_barrier()

    for b in range(NIB):
        pltpu.sync_copy(src_hbm.at[w, b], src_v)
        pltpu.sync_copy(dst_hbm.at[w, b], dst_v)
        pltpu.async_copy(x_hbm.at[src_v.at[0]], rows_a, sem_a)

        def body(p, carry):
            j = 2 * p
            pltpu.make_async_copy(x_hbm.at[src_v.at[j]], rows_a, sem_a).wait()
            pltpu.async_copy(x_hbm.at[src_v.at[j + 1]], rows_b, sem_b)
            pltpu.sync_copy(rows_a, shared.at[dst_v.at[j]], add=True)
            pltpu.make_async_copy(x_hbm.at[src_v.at[j + 1]], rows_b, sem_b).wait()
            pltpu.async_copy(x_hbm.at[src_v.at[j + 2]], rows_a, sem_a)
            pltpu.sync_copy(rows_b, shared.at[dst_v.at[j + 1]], add=True)
            return carry
        lax.fori_loop(0, (IB - 1) // 2, body, 0)
        pltpu.make_async_copy(x_hbm.at[src_v.at[IB - 1]], rows_a, sem_a).wait()
        pltpu.sync_copy(rows_a, shared.at[dst_v.at[IB - 1]], add=True)
    plsc.subcore_barrier()

    for k in range(RPT // CH):
        pltpu.sync_copy(shared.at[pl.ds(s * RPT + k * CH, CH)], rows_a)
        pltpu.sync_copy(rows_a, out_hbm.at[c, pl.ds(s * RPT + k * CH, CH)])


_sc_agg = pl.kernel(
    _sc_agg_body,
    out_type=jax.ShapeDtypeStruct((NC, NP, D), jnp.float32),
    mesh=plsc.VectorSubcoreMesh(core_axis_name="c", subcore_axis_name="s"),
    scratch_types=[
        pltpu.VMEM_SHARED((NP, D), jnp.float32),
        pltpu.VMEM((IB, CH), jnp.int32),
        pltpu.VMEM((IB, CH), jnp.int32),
        pltpu.VMEM((CH, D), jnp.float32),
        pltpu.VMEM((CH, D), jnp.float32),
        pltpu.SemaphoreType.DMA,
        pltpu.SemaphoreType.DMA,
    ],
)


def _mlp_body(eps_ref, x_ref, p_ref, wa_ref, ba_ref, wb_ref, bb_ref, out_ref):
    e = eps_ref[0, 0]
    s = e * x_ref[...] + p_ref[0] + p_ref[1]
    t = jnp.dot(s, wa_ref[...], preferred_element_type=jnp.float32) + ba_ref[...]
    t = jnp.maximum(t, 0.0)
    u = jnp.dot(t, wb_ref[...], preferred_element_type=jnp.float32) + bb_ref[...]
    out_ref[...] = jnp.maximum(u, 0.0)


def _mlp(e, x, p, wa, ba, wb, bb):
    return pl.pallas_call(
        _mlp_body,
        grid=(NRB,),
        in_specs=[
            pl.BlockSpec((1, 1), lambda i: (0, 0)),
            pl.BlockSpec((RB, D), lambda i: (i, 0)),
            pl.BlockSpec((NC, RB, D), lambda i: (0, i, 0)),
            pl.BlockSpec((D, D), lambda i: (0, 0)),
            pl.BlockSpec((1, D), lambda i: (0, 0)),
            pl.BlockSpec((D, D), lambda i: (0, 0)),
            pl.BlockSpec((1, D), lambda i: (0, 0)),
        ],
        out_specs=pl.BlockSpec((RB, D), lambda i: (i, 0)),
        out_shape=jax.ShapeDtypeStruct((N, D), jnp.float32),
    )(e, x, p, wa, ba, wb, bb)


def _fin_body(eps_ref, x_ref, p_ref, wa_ref, ba_ref, wb_ref, bb_ref,
              batch_ref, wc_ref, bc_ref, out_ref, acc_ref):
    i = pl.program_id(0)
    e = eps_ref[0, 0]
    s = e * x_ref[...] + p_ref[0] + p_ref[1]
    t = jnp.dot(s, wa_ref[...], preferred_element_type=jnp.float32) + ba_ref[...]
    t = jnp.maximum(t, 0.0)
    u = jnp.dot(t, wb_ref[...], preferred_element_type=jnp.float32) + bb_ref[...]
    u = jnp.maximum(u, 0.0)
    bvec = batch_ref[0, 0, :]
    oh = (bvec[:, None] == lax.broadcasted_iota(jnp.int32, (RB, G), 1))
    part = lax.dot_general(oh.astype(jnp.float32), u,
                           (((0,), (0,)), ((), ())),
                           preferred_element_type=jnp.float32)

    @pl.when(i == 0)
    def _():
        acc_ref[...] = part

    @pl.when(i > 0)
    def _():
        acc_ref[...] = acc_ref[...] + part

    @pl.when(i == NRB - 1)
    def _():
        out_ref[...] = (jnp.dot(acc_ref[...], wc_ref[...],
                                preferred_element_type=jnp.float32)
                        + bc_ref[...])


def _fin(e, x, p, wa, ba, wb, bb, batch3, wc, bc):
    o = wb.shape[1]
    c = wc.shape[1]
    return pl.pallas_call(
        _fin_body,
        grid=(NRB,),
        in_specs=[
            pl.BlockSpec((1, 1), lambda i: (0, 0)),
            pl.BlockSpec((RB, D), lambda i: (i, 0)),
            pl.BlockSpec((NC, RB, D), lambda i: (0, i, 0)),
            pl.BlockSpec((D, D), lambda i: (0, 0)),
            pl.BlockSpec((1, D), lambda i: (0, 0)),
            pl.BlockSpec((D, o), lambda i: (0, 0)),
            pl.BlockSpec((1, o), lambda i: (0, 0)),
            pl.BlockSpec((1, 1, RB), lambda i: (i, 0, 0)),
            pl.BlockSpec((o, c), lambda i: (0, 0)),
            pl.BlockSpec((1, c), lambda i: (0, 0)),
        ],
        out_specs=pl.BlockSpec((G, c), lambda i: (0, 0)),
        out_shape=jax.ShapeDtypeStruct((G, c), jnp.float32),
        scratch_shapes=[pltpu.VMEM((G, o), jnp.float32)],
    )(e, x, p, wa, ba, wb, bb, batch3, wc, bc)


def _fold_bn(wa, ba, g, be, rm, rv):
    scale = g / jnp.sqrt(rv + 1e-5)
    return wa * scale[None, :], (ba - rm) * scale + be


def kernel(x, edge_index, batch,
           eps1, w1a, b1a, g1, be1, rm1, rv1, w1b, b1b,
           eps2, w2a, b2a, g2, be2, rm2, rv2, w2b, b2b,
           eps3, w3a, b3a, g3, be3, rm3, rv3, w3b, b3b,
           wc, bc):
    src3 = edge_index[0].reshape(NW, NIB, IB, CH)
    dst3 = edge_index[1].reshape(NW, NIB, IB, CH)
    batch3 = batch.reshape(NRB, 1, RB)

    w1a_, b1a_ = _fold_bn(w1a, b1a, g1, be1, rm1, rv1)
    w2a_, b2a_ = _fold_bn(w2a, b2a, g2, be2, rm2, rv2)
    w3a_, b3a_ = _fold_bn(w3a, b3a, g3, be3, rm3, rv3)

    e1 = (1.0 + eps1).reshape(1, 1)
    e2 = (1.0 + eps2).reshape(1, 1)
    e3 = (1.0 + eps3).reshape(1, 1)

    p = _sc_agg(x, src3, dst3)
    h = _mlp(e1, x, p, w1a_, b1a_.reshape(1, D), w1b, b1b.reshape(1, D))
    p = _sc_agg(h, src3, dst3)
    h = _mlp(e2, h, p, w2a_, b2a_.reshape(1, D), w2b, b2b.reshape(1, D))
    p = _sc_agg(h, src3, dst3)
    return _fin(e3, h, p, w3a_, b3a_.reshape(1, D), w3b,
                b3b.reshape(1, w3b.shape[1]), batch3, wc, bc.reshape(1, 2))


# speedup vs baseline: 10.2543x; 1.2405x over previous
import jax
import jax.numpy as jnp
from jax import lax
from jax.experimental import pallas as pl
from jax.experimental.pallas import tpu as pltpu
from jax.experimental.pallas import tpu_sc as plsc

N = 10000
E = 320000
D = 128
G = 64

NC = 2
NS = 16
NW = NC * NS
EPW = E // NW
CH = 40
NBUF = 5
IB = 50
NIB = EPW // (IB * CH)
NGB = IB // NBUF
NP = 10240
RPT = NP // NS

RB = 1000
NRB = N // RB


def _sc_agg_body(x_hbm, src_hbm, dst_hbm, out_hbm, shared,
                 src_v, dst_v, r0, r1, r2, r3, r4,
                 g0, g1, g2, g3, g4, s0, s1, s2, s3, s4):
    c = lax.axis_index("c")
    s = lax.axis_index("s")
    w = c * NS + s
    bufs = (r0, r1, r2, r3, r4)
    gsems = (g0, g1, g2, g3, g4)
    ssems = (s0, s1, s2, s3, s4)

    def zrow(i, carry):
        for lb in range(D // 16):
            r0[i, pl.ds(lb * 16, 16)] = jnp.zeros((16,), jnp.float32)
        return carry
    lax.fori_loop(0, CH, zrow, 0)
    for k in range(RPT // CH):
        pltpu.sync_copy(r0, shared.at[pl.ds(s * RPT + k * CH, CH)])
    plsc.subcore_barrier()

    for b in range(NIB):
        pltpu.sync_copy(src_hbm.at[w, b], src_v)
        pltpu.sync_copy(dst_hbm.at[w, b], dst_v)
        for i in range(NBUF):
            pltpu.async_copy(x_hbm.at[src_v.at[i]], bufs[i], gsems[i])

        def gbody(t, carry):
            j = NBUF * t
            scat = []
            for i in range(NBUF):
                pltpu.make_async_copy(
                    x_hbm.at[src_v.at[j + i]], bufs[i], gsems[i]).wait()
                scat.append(pltpu.async_copy(
                    bufs[i], shared.at[dst_v.at[j + i]], ssems[i], add=True))
            for i in range(NBUF):
                scat[i].wait()
                pltpu.async_copy(
                    x_hbm.at[src_v.at[j + NBUF + i]], bufs[i], gsems[i])
            return carry
        lax.fori_loop(0, NGB - 1, gbody, 0)

        jlast = (NGB - 1) * NBUF
        scat = []
        for i in range(NBUF):
            pltpu.make_async_copy(
                x_hbm.at[src_v.at[jlast + i]], bufs[i], gsems[i]).wait()
            scat.append(pltpu.async_copy(
                bufs[i], shared.at[dst_v.at[jlast + i]], ssems[i], add=True))
        for i in range(NBUF):
            scat[i].wait()
    plsc.subcore_barrier()

    for k in range(RPT // CH):
        pltpu.sync_copy(shared.at[pl.ds(s * RPT + k * CH, CH)], r0)
        pltpu.sync_copy(r0, out_hbm.at[c, pl.ds(s * RPT + k * CH, CH)])


_sc_agg = pl.kernel(
    _sc_agg_body,
    out_type=jax.ShapeDtypeStruct((NC, NP, D), jnp.float32),
    mesh=plsc.VectorSubcoreMesh(core_axis_name="c", subcore_axis_name="s"),
    scratch_types=[
        pltpu.VMEM_SHARED((NP, D), jnp.float32),
        pltpu.VMEM((IB, CH), jnp.int32),
        pltpu.VMEM((IB, CH), jnp.int32),
        pltpu.VMEM((CH, D), jnp.float32),
        pltpu.VMEM((CH, D), jnp.float32),
        pltpu.VMEM((CH, D), jnp.float32),
        pltpu.VMEM((CH, D), jnp.float32),
        pltpu.VMEM((CH, D), jnp.float32),
        pltpu.SemaphoreType.DMA,
        pltpu.SemaphoreType.DMA,
        pltpu.SemaphoreType.DMA,
        pltpu.SemaphoreType.DMA,
        pltpu.SemaphoreType.DMA,
        pltpu.SemaphoreType.DMA,
        pltpu.SemaphoreType.DMA,
        pltpu.SemaphoreType.DMA,
        pltpu.SemaphoreType.DMA,
        pltpu.SemaphoreType.DMA,
    ],
)


def _mlp_body(eps_ref, x_ref, p_ref, wa_ref, ba_ref, wb_ref, bb_ref, out_ref):
    e = eps_ref[0, 0]
    s = e * x_ref[...] + p_ref[0] + p_ref[1]
    t = jnp.dot(s, wa_ref[...], preferred_element_type=jnp.float32) + ba_ref[...]
    t = jnp.maximum(t, 0.0)
    u = jnp.dot(t, wb_ref[...], preferred_element_type=jnp.float32) + bb_ref[...]
    out_ref[...] = jnp.maximum(u, 0.0)


def _mlp(e, x, p, wa, ba, wb, bb):
    return pl.pallas_call(
        _mlp_body,
        grid=(NRB,),
        in_specs=[
            pl.BlockSpec((1, 1), lambda i: (0, 0)),
            pl.BlockSpec((RB, D), lambda i: (i, 0)),
            pl.BlockSpec((NC, RB, D), lambda i: (0, i, 0)),
            pl.BlockSpec((D, D), lambda i: (0, 0)),
            pl.BlockSpec((1, D), lambda i: (0, 0)),
            pl.BlockSpec((D, D), lambda i: (0, 0)),
            pl.BlockSpec((1, D), lambda i: (0, 0)),
        ],
        out_specs=pl.BlockSpec((RB, D), lambda i: (i, 0)),
        out_shape=jax.ShapeDtypeStruct((N, D), jnp.float32),
    )(e, x, p, wa, ba, wb, bb)


def _fin_body(eps_ref, x_ref, p_ref, wa_ref, ba_ref, wb_ref, bb_ref,
              batch_ref, wc_ref, bc_ref, out_ref, acc_ref):
    i = pl.program_id(0)
    e = eps_ref[0, 0]
    s = e * x_ref[...] + p_ref[0] + p_ref[1]
    t = jnp.dot(s, wa_ref[...], preferred_element_type=jnp.float32) + ba_ref[...]
    t = jnp.maximum(t, 0.0)
    u = jnp.dot(t, wb_ref[...], preferred_element_type=jnp.float32) + bb_ref[...]
    u = jnp.maximum(u, 0.0)
    bvec = batch_ref[0, 0, :]
    oh = (bvec[:, None] == lax.broadcasted_iota(jnp.int32, (RB, G), 1))
    part = lax.dot_general(oh.astype(jnp.float32), u,
                           (((0,), (0,)), ((), ())),
                           preferred_element_type=jnp.float32)

    @pl.when(i == 0)
    def _():
        acc_ref[...] = part

    @pl.when(i > 0)
    def _():
        acc_ref[...] = acc_ref[...] + part

    @pl.when(i == NRB - 1)
    def _():
        out_ref[...] = (jnp.dot(acc_ref[...], wc_ref[...],
                                preferred_element_type=jnp.float32)
                        + bc_ref[...])


def _fin(e, x, p, wa, ba, wb, bb, batch3, wc, bc):
    o = wb.shape[1]
    cc = wc.shape[1]
    return pl.pallas_call(
        _fin_body,
        grid=(NRB,),
        in_specs=[
            pl.BlockSpec((1, 1), lambda i: (0, 0)),
            pl.BlockSpec((RB, D), lambda i: (i, 0)),
            pl.BlockSpec((NC, RB, D), lambda i: (0, i, 0)),
            pl.BlockSpec((D, D), lambda i: (0, 0)),
            pl.BlockSpec((1, D), lambda i: (0, 0)),
            pl.BlockSpec((D, o), lambda i: (0, 0)),
            pl.BlockSpec((1, o), lambda i: (0, 0)),
            pl.BlockSpec((1, 1, RB), lambda i: (i, 0, 0)),
            pl.BlockSpec((o, cc), lambda i: (0, 0)),
            pl.BlockSpec((1, cc), lambda i: (0, 0)),
        ],
        out_specs=pl.BlockSpec((G, cc), lambda i: (0, 0)),
        out_shape=jax.ShapeDtypeStruct((G, cc), jnp.float32),
        scratch_shapes=[pltpu.VMEM((G, o), jnp.float32)],
    )(e, x, p, wa, ba, wb, bb, batch3, wc, bc)


def _fold_bn(wa, ba, g, be, rm, rv):
    scale = g / jnp.sqrt(rv + 1e-5)
    return wa * scale[None, :], (ba - rm) * scale + be


def kernel(x, edge_index, batch,
           eps1, w1a, b1a, g1, be1, rm1, rv1, w1b, b1b,
           eps2, w2a, b2a, g2, be2, rm2, rv2, w2b, b2b,
           eps3, w3a, b3a, g3, be3, rm3, rv3, w3b, b3b,
           wc, bc):
    src4 = edge_index[0].reshape(NW, NIB, IB, CH)
    dst4 = edge_index[1].reshape(NW, NIB, IB, CH)
    batch3 = batch.reshape(NRB, 1, RB)

    w1a_, b1a_ = _fold_bn(w1a, b1a, g1, be1, rm1, rv1)
    w2a_, b2a_ = _fold_bn(w2a, b2a, g2, be2, rm2, rv2)
    w3a_, b3a_ = _fold_bn(w3a, b3a, g3, be3, rm3, rv3)

    e1 = (1.0 + eps1).reshape(1, 1)
    e2 = (1.0 + eps2).reshape(1, 1)
    e3 = (1.0 + eps3).reshape(1, 1)

    p = _sc_agg(x, src4, dst4)
    h = _mlp(e1, x, p, w1a_, b1a_.reshape(1, D), w1b, b1b.reshape(1, D))
    p = _sc_agg(h, src4, dst4)
    h = _mlp(e2, h, p, w2a_, b2a_.reshape(1, D), w2b, b2b.reshape(1, D))
    p = _sc_agg(h, src4, dst4)
    return _fin(e3, h, p, w3a_, b3a_.reshape(1, D), w3b,
                b3b.reshape(1, w3b.shape[1]), batch3, wc, bc.reshape(1, 2))
